# baseline (speedup 1.0000x reference)
import functools

import jax
import jax.numpy as jnp
from jax import lax
from jax.experimental import pallas as pl
from jax.experimental.pallas import tpu as pltpu
from jax.experimental.pallas import tpu_sc as plsc

N_USER = 10000
N_ITEM = 50000
E = 320000
D = 128
H = D // 2
B = 4096

NC = 2
NS = 16
NW = NC * NS
EPT = E // NS
K = 200
STRIPE = 624
STRIPE_LAST = N_USER - (NS - 1) * STRIPE
GPT = B // NW

_MESH = dict(core_axis_name="c", subcore_axis_name="s")


def _zero_stripe(zf, sh_S, s):
    @pl.when(s < NS - 1)
    def _():
        pltpu.sync_copy(zf.at[pl.ds(0, STRIPE)],
                        sh_S.at[pl.ds(s * STRIPE, STRIPE)])

    @pl.when(s == NS - 1)
    def _():
        pltpu.sync_copy(zf, sh_S.at[pl.ds((NS - 1) * STRIPE, STRIPE_LAST)])


def _stage_table(hX, sh_T, s):
    @pl.when(s < NS - 1)
    def _():
        pltpu.sync_copy(hX.at[pl.ds(s * STRIPE, STRIPE)],
                        sh_T.at[pl.ds(s * STRIPE, STRIPE)])

    @pl.when(s == NS - 1)
    def _():
        pltpu.sync_copy(hX.at[pl.ds((NS - 1) * STRIPE, STRIPE_LAST)],
                        sh_T.at[pl.ds((NS - 1) * STRIPE, STRIPE_LAST)])


def _write_stripes(sh_S, S_out, s, co):
    @pl.when(s < NS - 1)
    def _():
        pltpu.sync_copy(sh_S.at[pl.ds(s * STRIPE, STRIPE)],
                        S_out.at[pl.ds(s * STRIPE, STRIPE), pl.ds(co, H)])

    @pl.when(s == NS - 1)
    def _():
        pltpu.sync_copy(
            sh_S.at[pl.ds((NS - 1) * STRIPE, STRIPE_LAST)],
            S_out.at[pl.ds((NS - 1) * STRIPE, STRIPE_LAST), pl.ds(co, H)])


def _fill_ones(ones_ref):
    def body(i, _):
        ones_ref[pl.ds(i * 16, 16)] = jnp.ones((16,), jnp.float32)
        return 0
    lax.fori_loop(0, K // 16, body, 0)


NCH = EPT // K


def _edge_pipeline(h, src, dst, s, c, sh_S,
                   sem_g, sem_s, sem_si, sem_di,
                   si0, di0, rows0, si1, di1, rows1,
                   ones=None, sh_deg=None, sem_o=None):
    base_e = s * EPT

    def li_start(i, sidx):
        pltpu.async_copy(src.at[pl.ds(base_e + i * K, K)], sidx, sem_si)

    def li_wait(sidx):
        pltpu.make_async_copy(src.at[pl.ds(base_e, K)], sidx, sem_si).wait()

    def ld_start(i, didx):
        pltpu.async_copy(dst.at[pl.ds(base_e + i * K, K)], didx, sem_di)

    def ld_wait(didx):
        pltpu.make_async_copy(dst.at[pl.ds(base_e, K)], didx, sem_di).wait()

    def deg_start(didx, core):
        if sh_deg is not None:
            @pl.when(c == core)
            def _():
                pltpu.async_copy(ones, sh_deg.at[didx], sem_o, add=True)

    def deg_wait(didx, core):
        if sh_deg is not None:
            @pl.when(c == core)
            def _():
                pltpu.make_async_copy(ones, sh_deg.at[didx], sem_o).wait()

    li_start(0, si0)
    ld_start(0, di0)
    li_wait(si0)
    pltpu.async_copy(h.at[si0], rows0, sem_g)
    li_start(1, si1)

    NJ = NCH // 2

    def pair(j, _):
        pltpu.make_async_copy(h.at[si0], rows0, sem_g).wait()
        ld_wait(di0)
        pltpu.async_copy(rows0, sh_S.at[di0], sem_s, add=True)
        deg_start(di0, 0)

        @pl.when(j > 0)
        def _():
            pltpu.make_async_copy(rows1, sh_S.at[di1], sem_s).wait()
            deg_wait(di1, 1)

        ld_start(2 * j + 1, di1)
        li_wait(si1)
        pltpu.async_copy(h.at[si1], rows1, sem_g)

        @pl.when(j < NJ - 1)
        def _():
            li_start(2 * j + 2, si0)

        pltpu.make_async_copy(h.at[si1], rows1, sem_g).wait()
        ld_wait(di1)
        pltpu.async_copy(rows1, sh_S.at[di1], sem_s, add=True)
        deg_start(di1, 1)

        pltpu.make_async_copy(rows0, sh_S.at[di0], sem_s).wait()
        deg_wait(di0, 0)

        @pl.when(j < NJ - 1)
        def _():
            ld_start(2 * j + 2, di0)
            li_wait(si0)
            pltpu.async_copy(h.at[si0], rows0, sem_g)
            li_start(2 * j + 3, si1)

        return 0

    lax.fori_loop(0, NJ, pair, 0)
    pltpu.make_async_copy(rows1, sh_S.at[di1], sem_s).wait()
    deg_wait(di1, 1)


def _gather_rows(table, idx_hbm, out_hbm, gidx, grows, sem, wid):
    gb = wid * GPT
    pltpu.sync_copy(idx_hbm.at[pl.ds(gb, GPT)], gidx)
    pltpu.async_copy(table.at[gidx], grows, sem).wait()
    pltpu.sync_copy(grows, out_hbm.at[pl.ds(gb, GPT)])


_EDGE_SCRATCH = [
    pltpu.VMEM((K,), jnp.int32),
    pltpu.VMEM((K,), jnp.int32),
    pltpu.VMEM((K, H), jnp.float32),
    pltpu.VMEM((K,), jnp.int32),
    pltpu.VMEM((K,), jnp.int32),
    pltpu.VMEM((K, H), jnp.float32),
    pltpu.SemaphoreType.DMA,
    pltpu.SemaphoreType.DMA,
    pltpu.SemaphoreType.DMA,
    pltpu.SemaphoreType.DMA,
]


@functools.partial(
    pl.kernel,
    out_type=(
        jax.ShapeDtypeStruct((N_USER, D), jnp.float32),
        jax.ShapeDtypeStruct((NC * N_USER,), jnp.float32),
    ),
    mesh=plsc.VectorSubcoreMesh(**_MESH),
    compiler_params=pltpu.CompilerParams(use_tc_tiling_on_sc=False),
    scratch_types=_EDGE_SCRATCH + [
        pltpu.VMEM((K,), jnp.float32),
        pltpu.VMEM((N_USER,), jnp.float32),
        pltpu.VMEM_SHARED((N_USER, H), jnp.float32),
        pltpu.VMEM_SHARED((N_USER, H), jnp.float32),
        pltpu.VMEM_SHARED((N_USER,), jnp.float32),
        pltpu.SemaphoreType.DMA,
    ],
)
def _sc_layer1(hL, hR, src, dst, zf,
               S_out, deg_out,
               si0, di0, rows0, si1, di1, rows1,
               sem_g, sem_s, sem_si, sem_di,
               ones, deg_v, sh_S, sh_T, sh_deg, sem_o):
    c = lax.axis_index("c")
    s = lax.axis_index("s")
    co = c * H

    _zero_stripe(zf, sh_S, s)

    @pl.when(c == 0)
    def _():
        _stage_table(hL, sh_T, s)

    @pl.when(c == 1)
    def _():
        _stage_table(hR, sh_T, s)

    @pl.when(s == 0)
    def _():
        def zbody(i, _):
            deg_v[pl.ds(i * 16, 16)] = jnp.zeros((16,), jnp.float32)
            return 0
        lax.fori_loop(0, N_USER // 16, zbody, 0)
        pltpu.sync_copy(deg_v, sh_deg)

    _fill_ones(ones)
    plsc.subcore_barrier()

    _edge_pipeline(sh_T, src, dst, s, c, sh_S,
                   sem_g, sem_s, sem_si, sem_di,
                   si0, di0, rows0, si1, di1, rows1,
                   ones=ones, sh_deg=sh_deg, sem_o=sem_o)

    plsc.subcore_barrier()
    _write_stripes(sh_S, S_out, s, co)

    @pl.when(s == 0)
    def _():
        pltpu.sync_copy(sh_deg, deg_v)
        pltpu.sync_copy(deg_v, deg_out.at[pl.ds(c * N_USER, N_USER)])


@functools.partial(
    pl.kernel,
    out_type=(
        jax.ShapeDtypeStruct((N_USER, D), jnp.float32),
        jax.ShapeDtypeStruct((B, D), jnp.float32),
    ),
    mesh=plsc.VectorSubcoreMesh(**_MESH),
    compiler_params=pltpu.CompilerParams(use_tc_tiling_on_sc=False),
    scratch_types=_EDGE_SCRATCH + [
        pltpu.VMEM((GPT,), jnp.int32),
        pltpu.VMEM((GPT, D), jnp.float32),
        pltpu.VMEM_SHARED((N_USER, H), jnp.float32),
        pltpu.VMEM_SHARED((N_USER, H), jnp.float32),
        pltpu.SemaphoreType.DMA,
    ],
)
def _sc_layer2(h_full, hL, hR, src, dst, users, zf,
               S_out, gu,
               si0, di0, rows0, si1, di1, rows1,
               sem_g, sem_s, sem_si, sem_di,
               gidx, grows, sh_S, sh_T, sem_u):
    c = lax.axis_index("c")
    s = lax.axis_index("s")
    co = c * H
    wid = c * NS + s

    _zero_stripe(zf, sh_S, s)

    @pl.when(c == 0)
    def _():
        _stage_table(hL, sh_T, s)

    @pl.when(c == 1)
    def _():
        _stage_table(hR, sh_T, s)

    plsc.subcore_barrier()

    _edge_pipeline(sh_T, src, dst, s, c, sh_S,
                   sem_g, sem_s, sem_si, sem_di,
                   si0, di0, rows0, si1, di1, rows1)

    _gather_rows(h_full, users, gu, gidx, grows, sem_u, wid)

    plsc.subcore_barrier()
    _write_stripes(sh_S, S_out, s, co)


@functools.partial(
    pl.kernel,
    out_type=jax.ShapeDtypeStruct((B, D), jnp.float32),
    mesh=plsc.VectorSubcoreMesh(**_MESH),
    scratch_types=[
        pltpu.VMEM((GPT,), jnp.int32),
        pltpu.VMEM((GPT, D), jnp.float32),
        pltpu.SemaphoreType.DMA,
    ],
)
def _sc_gather(table, idx, out, gidx, grows, sem):
    c = lax.axis_index("c")
    s = lax.axis_index("s")
    wid = c * NS + s
    _gather_rows(table, idx, out, gidx, grows, sem, wid)


@functools.partial(
    pl.kernel,
    out_type=(
        jax.ShapeDtypeStruct((B, D), jnp.float32),
        jax.ShapeDtypeStruct((B, D), jnp.float32),
        jax.ShapeDtypeStruct((B, D), jnp.float32),
    ),
    mesh=plsc.VectorSubcoreMesh(**_MESH),
    scratch_types=[
        pltpu.VMEM((GPT,), jnp.int32),
        pltpu.VMEM((GPT, D), jnp.float32),
        pltpu.SemaphoreType.DMA,
    ],
)
def _sc_gather3(fuser, fitem, users, pos, neg, gu, gp, gn, gidx, grows, sem):
    c = lax.axis_index("c")
    s = lax.axis_index("s")
    wid = c * NS + s
    _gather_rows(fuser, users, gu, gidx, grows, sem, wid)
    _gather_rows(fitem, pos, gp, gidx, grows, sem, wid)
    _gather_rows(fitem, neg, gn, gidx, grows, sem, wid)


def _leaky_l2(z):
    z = jnp.where(z >= 0, z, 0.2 * z)
    n2 = jnp.sum(z * z, axis=1, keepdims=True)
    return z * lax.rsqrt(jnp.maximum(n2, 1e-24))


def _sage_body(h_ref, S_ref, deg_ref, Ws_ref, Wn_ref, b_ref, o_ref):
    deg = jnp.sum(deg_ref[...], axis=1, keepdims=True)
    inv = 1.0 / jnp.maximum(deg, 1.0)
    z = (jnp.dot(h_ref[...], Ws_ref[...], preferred_element_type=jnp.float32)
         + jnp.dot(S_ref[...] * inv, Wn_ref[...],
                   preferred_element_type=jnp.float32)
         + b_ref[...])
    o_ref[...] = _leaky_l2(z)


_SAGE_RB = 2000


def _sage_tc(h, S, deg2, Ws, Wn, b2):
    return pl.pallas_call(
        _sage_body,
        grid=(N_USER // _SAGE_RB,),
        in_specs=[
            pl.BlockSpec((_SAGE_RB, D), lambda i: (i, 0)),
            pl.BlockSpec((_SAGE_RB, D), lambda i: (i, 0)),
            pl.BlockSpec((_SAGE_RB, NC), lambda i: (i, 0)),
            pl.BlockSpec((D, D), lambda i: (0, 0)),
            pl.BlockSpec((D, D), lambda i: (0, 0)),
            pl.BlockSpec((1, D), lambda i: (0, 0)),
        ],
        out_specs=pl.BlockSpec((_SAGE_RB, D), lambda i: (i, 0)),
        out_shape=jax.ShapeDtypeStruct((N_USER, D), jnp.float32),
    )(h, S, deg2, Ws, Wn, b2)


def _item_body(x_ref, W0_ref, b0_ref, W1_ref, b1_ref, o1_ref, o2_ref):
    x1 = _leaky_l2(jnp.dot(x_ref[...], W0_ref[...],
                           preferred_element_type=jnp.float32) + b0_ref[...])
    o1_ref[...] = x1
    x2 = _leaky_l2(jnp.dot(x1, W1_ref[...],
                           preferred_element_type=jnp.float32) + b1_ref[...])
    o2_ref[...] = x2


_ITEM_RB = 1024


def _item_tc(x0, W0, b0, W1, b1):
    nb = (2 * B) // _ITEM_RB
    return pl.pallas_call(
        _item_body,
        grid=(nb,),
        in_specs=[
            pl.BlockSpec((_ITEM_RB, D), lambda i: (i, 0)),
            pl.BlockSpec((D, D), lambda i: (0, 0)),
            pl.BlockSpec((1, D), lambda i: (0, 0)),
            pl.BlockSpec((D, D), lambda i: (0, 0)),
            pl.BlockSpec((1, D), lambda i: (0, 0)),
        ],
        out_specs=[
            pl.BlockSpec((_ITEM_RB, D), lambda i: (i, 0)),
            pl.BlockSpec((_ITEM_RB, D), lambda i: (i, 0)),
        ],
        out_shape=[
            jax.ShapeDtypeStruct((2 * B, D), jnp.float32),
            jax.ShapeDtypeStruct((2 * B, D), jnp.float32),
        ],
    )(x0, W0, b0, W1, b1)


def kernel(users, pos_items, neg_items, edge_index, feat_user, feat_item,
           W_self_0, W_neigh_0, b_0, W_self_1, W_neigh_1, b_1,
           Wi_0, bi_0, Wi_1, bi_1):
    src = edge_index[0].astype(jnp.int32)
    dst = edge_index[1].astype(jnp.int32)
    users = users.astype(jnp.int32)
    pos_items = pos_items.astype(jnp.int32)
    neg_items = neg_items.astype(jnp.int32)

    zf = jnp.zeros((STRIPE_LAST, H), jnp.float32)

    S0, deg = _sc_layer1(feat_user[:, :H], feat_user[:, H:], src, dst, zf)
    gu0, gp0, gn0 = _sc_gather3(feat_user, feat_item, users, pos_items,
                                neg_items)
    deg2 = deg.reshape(NC, N_USER).T

    h1 = _sage_tc(feat_user, S0, deg2, W_self_0, W_neigh_0, b_0.reshape(1, D))
    S1, gu1 = _sc_layer2(h1, h1[:, :H], h1[:, H:], src, dst, users, zf)
    h2 = _sage_tc(h1, S1, deg2, W_self_1, W_neigh_1, b_1.reshape(1, D))
    gu2 = _sc_gather(h2, users)

    x0 = jnp.concatenate([gp0, gn0], axis=0)
    x1, x2 = _item_tc(x0, Wi_0, bi_0.reshape(1, D), Wi_1, bi_1.reshape(1, D))

    u_g = jnp.concatenate([gu0, gu1, gu2], axis=1)
    pos_i_g = jnp.concatenate([gp0, x1[:B], x2[:B]], axis=1)
    neg_i_g = jnp.concatenate([gn0, x1[B:], x2[B:]], axis=1)
    return (u_g, pos_i_g, neg_i_g)


# speedup vs baseline: 1.2543x; 1.2543x over previous
import functools

import jax
import jax.numpy as jnp
from jax import lax
from jax.experimental import pallas as pl
from jax.experimental.pallas import tpu as pltpu
from jax.experimental.pallas import tpu_sc as plsc

N_USER = 10000
N_ITEM = 50000
E = 320000
D = 128
H = D // 2
B = 4096

NC = 2
NS = 16
NW = NC * NS
EPT = E // NS
K = 400
STRIPE = 624
STRIPE_LAST = N_USER - (NS - 1) * STRIPE
GPT = B // NW

_MESH = dict(core_axis_name="c", subcore_axis_name="s")


def _zero_stripe(zf, sh_S, s):
    @pl.when(s < NS - 1)
    def _():
        pltpu.sync_copy(zf.at[pl.ds(0, STRIPE)],
                        sh_S.at[pl.ds(s * STRIPE, STRIPE)])

    @pl.when(s == NS - 1)
    def _():
        pltpu.sync_copy(zf, sh_S.at[pl.ds((NS - 1) * STRIPE, STRIPE_LAST)])


def _write_stripes(sh_S, S_out, s, co):
    @pl.when(s < NS - 1)
    def _():
        pltpu.sync_copy(sh_S.at[pl.ds(s * STRIPE, STRIPE)],
                        S_out.at[pl.ds(s * STRIPE, STRIPE), pl.ds(co, H)])

    @pl.when(s == NS - 1)
    def _():
        pltpu.sync_copy(
            sh_S.at[pl.ds((NS - 1) * STRIPE, STRIPE_LAST)],
            S_out.at[pl.ds((NS - 1) * STRIPE, STRIPE_LAST), pl.ds(co, H)])


def _fill_ones(ones_ref):
    def body(i, _):
        ones_ref[pl.ds(i * 16, 16)] = jnp.ones((16,), jnp.float32)
        return 0
    lax.fori_loop(0, K // 16, body, 0)


NCH = EPT // K


def _edge_pipeline(h, src, dst, s, c, sh_S,
                   sem_g, sem_s, sem_si, sem_di,
                   si0, di0, rows0, si1, di1, rows1,
                   ones=None, sh_deg=None, sem_o=None):
    base_e = s * EPT

    def li_start(i, sidx):
        pltpu.async_copy(src.at[pl.ds(base_e + i * K, K)], sidx, sem_si)

    def li_wait(sidx):
        pltpu.make_async_copy(src.at[pl.ds(base_e, K)], sidx, sem_si).wait()

    def ld_start(i, didx):
        pltpu.async_copy(dst.at[pl.ds(base_e + i * K, K)], didx, sem_di)

    def ld_wait(didx):
        pltpu.make_async_copy(dst.at[pl.ds(base_e, K)], didx, sem_di).wait()

    def deg_start(didx, core):
        if sh_deg is not None:
            @pl.when(c == core)
            def _():
                pltpu.async_copy(ones, sh_deg.at[didx], sem_o, add=True)

    def deg_wait(didx, core):
        if sh_deg is not None:
            @pl.when(c == core)
            def _():
                pltpu.make_async_copy(ones, sh_deg.at[didx], sem_o).wait()

    li_start(0, si0)
    ld_start(0, di0)
    li_wait(si0)
    pltpu.async_copy(h.at[si0], rows0, sem_g)
    li_start(1, si1)

    NJ = NCH // 2

    def pair(j, _):
        pltpu.make_async_copy(h.at[si0], rows0, sem_g).wait()
        ld_wait(di0)
        pltpu.async_copy(rows0, sh_S.at[di0], sem_s, add=True)
        deg_start(di0, 0)

        @pl.when(j > 0)
        def _():
            pltpu.make_async_copy(rows1, sh_S.at[di1], sem_s).wait()
            deg_wait(di1, 1)

        ld_start(2 * j + 1, di1)
        li_wait(si1)
        pltpu.async_copy(h.at[si1], rows1, sem_g)

        @pl.when(j < NJ - 1)
        def _():
            li_start(2 * j + 2, si0)

        pltpu.make_async_copy(h.at[si1], rows1, sem_g).wait()
        ld_wait(di1)
        pltpu.async_copy(rows1, sh_S.at[di1], sem_s, add=True)
        deg_start(di1, 1)

        pltpu.make_async_copy(rows0, sh_S.at[di0], sem_s).wait()
        deg_wait(di0, 0)

        @pl.when(j < NJ - 1)
        def _():
            ld_start(2 * j + 2, di0)
            li_wait(si0)
            pltpu.async_copy(h.at[si0], rows0, sem_g)
            li_start(2 * j + 3, si1)

        return 0

    lax.fori_loop(0, NJ, pair, 0)
    pltpu.make_async_copy(rows1, sh_S.at[di1], sem_s).wait()
    deg_wait(di1, 1)


def _gather_rows(table, idx_hbm, out_hbm, gidx, grows, sem, wid):
    gb = wid * GPT
    pltpu.sync_copy(idx_hbm.at[pl.ds(gb, GPT)], gidx)
    pltpu.async_copy(table.at[gidx], grows, sem).wait()
    pltpu.sync_copy(grows, out_hbm.at[pl.ds(gb, GPT)])


_EDGE_SCRATCH = [
    pltpu.VMEM((K,), jnp.int32),
    pltpu.VMEM((K,), jnp.int32),
    pltpu.VMEM((K, H), jnp.float32),
    pltpu.VMEM((K,), jnp.int32),
    pltpu.VMEM((K,), jnp.int32),
    pltpu.VMEM((K, H), jnp.float32),
    pltpu.SemaphoreType.DMA,
    pltpu.SemaphoreType.DMA,
    pltpu.SemaphoreType.DMA,
    pltpu.SemaphoreType.DMA,
]


@functools.partial(
    pl.kernel,
    out_type=(
        jax.ShapeDtypeStruct((N_USER, D), jnp.float32),
        jax.ShapeDtypeStruct((NC * N_USER,), jnp.float32),
    ),
    mesh=plsc.VectorSubcoreMesh(**_MESH),
    compiler_params=pltpu.CompilerParams(use_tc_tiling_on_sc=False),
    scratch_types=_EDGE_SCRATCH + [
        pltpu.VMEM((K,), jnp.float32),
        pltpu.VMEM((N_USER,), jnp.float32),
        pltpu.VMEM_SHARED((N_USER, H), jnp.float32),
        pltpu.VMEM_SHARED((N_USER,), jnp.float32),
        pltpu.SemaphoreType.DMA,
    ],
)
def _sc_layer1(hL, hR, src, dst, zf,
               S_out, deg_out,
               si0, di0, rows0, si1, di1, rows1,
               sem_g, sem_s, sem_si, sem_di,
               ones, deg_v, sh_S, sh_deg, sem_o):
    c = lax.axis_index("c")
    s = lax.axis_index("s")
    co = c * H

    _zero_stripe(zf, sh_S, s)

    @pl.when(s == 0)
    def _():
        def zbody(i, _):
            deg_v[pl.ds(i * 16, 16)] = jnp.zeros((16,), jnp.float32)
            return 0
        lax.fori_loop(0, N_USER // 16, zbody, 0)
        pltpu.sync_copy(deg_v, sh_deg)

    _fill_ones(ones)
    plsc.subcore_barrier()

    @pl.when(c == 0)
    def _():
        _edge_pipeline(hL, src, dst, s, c, sh_S,
                       sem_g, sem_s, sem_si, sem_di,
                       si0, di0, rows0, si1, di1, rows1,
                       ones=ones, sh_deg=sh_deg, sem_o=sem_o)

    @pl.when(c == 1)
    def _():
        _edge_pipeline(hR, src, dst, s, c, sh_S,
                       sem_g, sem_s, sem_si, sem_di,
                       si0, di0, rows0, si1, di1, rows1,
                       ones=ones, sh_deg=sh_deg, sem_o=sem_o)

    plsc.subcore_barrier()
    _write_stripes(sh_S, S_out, s, co)

    @pl.when(s == 0)
    def _():
        pltpu.sync_copy(sh_deg, deg_v)
        pltpu.sync_copy(deg_v, deg_out.at[pl.ds(c * N_USER, N_USER)])


@functools.partial(
    pl.kernel,
    out_type=(
        jax.ShapeDtypeStruct((N_USER, D), jnp.float32),
        jax.ShapeDtypeStruct((B, D), jnp.float32),
    ),
    mesh=plsc.VectorSubcoreMesh(**_MESH),
    compiler_params=pltpu.CompilerParams(use_tc_tiling_on_sc=False),
    scratch_types=_EDGE_SCRATCH + [
        pltpu.VMEM((GPT,), jnp.int32),
        pltpu.VMEM((GPT, D), jnp.float32),
        pltpu.VMEM_SHARED((N_USER, H), jnp.float32),
        pltpu.SemaphoreType.DMA,
    ],
)
def _sc_layer2(h_full, hL, hR, src, dst, users, zf,
               S_out, gu,
               si0, di0, rows0, si1, di1, rows1,
               sem_g, sem_s, sem_si, sem_di,
               gidx, grows, sh_S, sem_u):
    c = lax.axis_index("c")
    s = lax.axis_index("s")
    co = c * H
    wid = c * NS + s

    _zero_stripe(zf, sh_S, s)
    plsc.subcore_barrier()

    @pl.when(c == 0)
    def _():
        _edge_pipeline(hL, src, dst, s, c, sh_S,
                       sem_g, sem_s, sem_si, sem_di,
                       si0, di0, rows0, si1, di1, rows1)

    @pl.when(c == 1)
    def _():
        _edge_pipeline(hR, src, dst, s, c, sh_S,
                       sem_g, sem_s, sem_si, sem_di,
                       si0, di0, rows0, si1, di1, rows1)

    _gather_rows(h_full, users, gu, gidx, grows, sem_u, wid)

    plsc.subcore_barrier()
    _write_stripes(sh_S, S_out, s, co)


@functools.partial(
    pl.kernel,
    out_type=jax.ShapeDtypeStruct((B, D), jnp.float32),
    mesh=plsc.VectorSubcoreMesh(**_MESH),
    scratch_types=[
        pltpu.VMEM((GPT,), jnp.int32),
        pltpu.VMEM((GPT, D), jnp.float32),
        pltpu.SemaphoreType.DMA,
    ],
)
def _sc_gather(table, idx, out, gidx, grows, sem):
    c = lax.axis_index("c")
    s = lax.axis_index("s")
    wid = c * NS + s
    _gather_rows(table, idx, out, gidx, grows, sem, wid)


@functools.partial(
    pl.kernel,
    out_type=(
        jax.ShapeDtypeStruct((B, D), jnp.float32),
        jax.ShapeDtypeStruct((B, D), jnp.float32),
        jax.ShapeDtypeStruct((B, D), jnp.float32),
    ),
    mesh=plsc.VectorSubcoreMesh(**_MESH),
    scratch_types=[
        pltpu.VMEM((GPT,), jnp.int32),
        pltpu.VMEM((GPT, D), jnp.float32),
        pltpu.SemaphoreType.DMA,
    ],
)
def _sc_gather3(fuser, fitem, users, pos, neg, gu, gp, gn, gidx, grows, sem):
    c = lax.axis_index("c")
    s = lax.axis_index("s")
    wid = c * NS + s
    _gather_rows(fuser, users, gu, gidx, grows, sem, wid)
    _gather_rows(fitem, pos, gp, gidx, grows, sem, wid)
    _gather_rows(fitem, neg, gn, gidx, grows, sem, wid)


def _leaky_l2(z):
    z = jnp.where(z >= 0, z, 0.2 * z)
    n2 = jnp.sum(z * z, axis=1, keepdims=True)
    return z * lax.rsqrt(jnp.maximum(n2, 1e-24))


def _sage_body(h_ref, S_ref, deg_ref, Ws_ref, Wn_ref, b_ref,
               o_ref, oL_ref, oR_ref):
    deg = jnp.sum(deg_ref[...], axis=1, keepdims=True)
    inv = 1.0 / jnp.maximum(deg, 1.0)
    z = (jnp.dot(h_ref[...], Ws_ref[...], preferred_element_type=jnp.float32)
         + jnp.dot(S_ref[...] * inv, Wn_ref[...],
                   preferred_element_type=jnp.float32)
         + b_ref[...])
    z = _leaky_l2(z)
    o_ref[...] = z
    oL_ref[...] = z[:, :H]
    oR_ref[...] = z[:, H:]


_SAGE_RB = 2000


def _sage_tc(h, S, deg2, Ws, Wn, b2):
    return pl.pallas_call(
        _sage_body,
        grid=(N_USER // _SAGE_RB,),
        in_specs=[
            pl.BlockSpec((_SAGE_RB, D), lambda i: (i, 0)),
            pl.BlockSpec((_SAGE_RB, D), lambda i: (i, 0)),
            pl.BlockSpec((_SAGE_RB, NC), lambda i: (i, 0)),
            pl.BlockSpec((D, D), lambda i: (0, 0)),
            pl.BlockSpec((D, D), lambda i: (0, 0)),
            pl.BlockSpec((1, D), lambda i: (0, 0)),
        ],
        out_specs=[
            pl.BlockSpec((_SAGE_RB, D), lambda i: (i, 0)),
            pl.BlockSpec((_SAGE_RB, H), lambda i: (i, 0)),
            pl.BlockSpec((_SAGE_RB, H), lambda i: (i, 0)),
        ],
        out_shape=[
            jax.ShapeDtypeStruct((N_USER, D), jnp.float32),
            jax.ShapeDtypeStruct((N_USER, H), jnp.float32),
            jax.ShapeDtypeStruct((N_USER, H), jnp.float32),
        ],
    )(h, S, deg2, Ws, Wn, b2)


def _item_body(x_ref, W0_ref, b0_ref, W1_ref, b1_ref, o_ref):
    x0 = x_ref[...]
    x1 = _leaky_l2(jnp.dot(x0, W0_ref[...],
                           preferred_element_type=jnp.float32) + b0_ref[...])
    x2 = _leaky_l2(jnp.dot(x1, W1_ref[...],
                           preferred_element_type=jnp.float32) + b1_ref[...])
    o_ref[:, 0:D] = x0
    o_ref[:, D:2 * D] = x1
    o_ref[:, 2 * D:3 * D] = x2


_ITEM_RB = 1024


def _item_tc(x0, W0, b0, W1, b1):
    return pl.pallas_call(
        _item_body,
        grid=(B // _ITEM_RB,),
        in_specs=[
            pl.BlockSpec((_ITEM_RB, D), lambda i: (i, 0)),
            pl.BlockSpec((D, D), lambda i: (0, 0)),
            pl.BlockSpec((1, D), lambda i: (0, 0)),
            pl.BlockSpec((D, D), lambda i: (0, 0)),
            pl.BlockSpec((1, D), lambda i: (0, 0)),
        ],
        out_specs=pl.BlockSpec((_ITEM_RB, 3 * D), lambda i: (i, 0)),
        out_shape=jax.ShapeDtypeStruct((B, 3 * D), jnp.float32),
    )(x0, W0, b0, W1, b1)


def kernel(users, pos_items, neg_items, edge_index, feat_user, feat_item,
           W_self_0, W_neigh_0, b_0, W_self_1, W_neigh_1, b_1,
           Wi_0, bi_0, Wi_1, bi_1):
    src = edge_index[0].astype(jnp.int32)
    dst = edge_index[1].astype(jnp.int32)
    users = users.astype(jnp.int32)
    pos_items = pos_items.astype(jnp.int32)
    neg_items = neg_items.astype(jnp.int32)

    zf = jnp.zeros((STRIPE_LAST, H), jnp.float32)

    S0, deg = _sc_layer1(feat_user[:, :H], feat_user[:, H:], src, dst, zf)
    gu0, gp0, gn0 = _sc_gather3(feat_user, feat_item, users, pos_items,
                                neg_items)
    deg2 = deg.reshape(NC, N_USER).T

    h1, h1L, h1R = _sage_tc(feat_user, S0, deg2,
                            W_self_0, W_neigh_0, b_0.reshape(1, D))
    S1, gu1 = _sc_layer2(h1, h1L, h1R, src, dst, users, zf)
    h2, _, _ = _sage_tc(h1, S1, deg2, W_self_1, W_neigh_1, b_1.reshape(1, D))
    gu2 = _sc_gather(h2, users)

    pos_i_g = _item_tc(gp0, Wi_0, bi_0.reshape(1, D), Wi_1, bi_1.reshape(1, D))
    neg_i_g = _item_tc(gn0, Wi_0, bi_0.reshape(1, D), Wi_1, bi_1.reshape(1, D))

    u_g = jnp.concatenate([gu0, gu1, gu2], axis=1)
    return (u_g, pos_i_g, neg_i_g)


# speedup vs baseline: 1.2660x; 1.0093x over previous
import functools

import jax
import jax.numpy as jnp
from jax import lax
from jax.experimental import pallas as pl
from jax.experimental.pallas import tpu as pltpu
from jax.experimental.pallas import tpu_sc as plsc

N_USER = 10000
N_ITEM = 50000
E = 320000
D = 128
H = D // 2
B = 4096

NC = 2
NS = 16
NW = NC * NS
EPT = E // NS
K = 400
STRIPE = 624
STRIPE_LAST = N_USER - (NS - 1) * STRIPE
GPT = B // NW

_MESH = dict(core_axis_name="c", subcore_axis_name="s")


def _zero_stripe(zf, sh_S, s):
    @pl.when(s < NS - 1)
    def _():
        pltpu.sync_copy(zf.at[pl.ds(0, STRIPE)],
                        sh_S.at[pl.ds(s * STRIPE, STRIPE)])

    @pl.when(s == NS - 1)
    def _():
        pltpu.sync_copy(zf, sh_S.at[pl.ds((NS - 1) * STRIPE, STRIPE_LAST)])


def _write_stripes(sh_S, S_out, s, co):
    @pl.when(s < NS - 1)
    def _():
        pltpu.sync_copy(sh_S.at[pl.ds(s * STRIPE, STRIPE)],
                        S_out.at[pl.ds(s * STRIPE, STRIPE), pl.ds(co, H)])

    @pl.when(s == NS - 1)
    def _():
        pltpu.sync_copy(
            sh_S.at[pl.ds((NS - 1) * STRIPE, STRIPE_LAST)],
            S_out.at[pl.ds((NS - 1) * STRIPE, STRIPE_LAST), pl.ds(co, H)])


def _fill_ones(ones_ref):
    def body(i, _):
        ones_ref[pl.ds(i * 16, 16)] = jnp.ones((16,), jnp.float32)
        return 0
    lax.fori_loop(0, K // 16, body, 0)


NCH = EPT // K


def _edge_pipeline(h, src, dst, s, c, sh_S,
                   sem_g, sem_s, sem_si, sem_di,
                   si0, di0, rows0, si1, di1, rows1,
                   ones=None, sh_deg=None, sem_o=None):
    base_e = s * EPT

    def li_start(i, sidx):
        pltpu.async_copy(src.at[pl.ds(base_e + i * K, K)], sidx, sem_si)

    def li_wait(sidx):
        pltpu.make_async_copy(src.at[pl.ds(base_e, K)], sidx, sem_si).wait()

    def ld_start(i, didx):
        pltpu.async_copy(dst.at[pl.ds(base_e + i * K, K)], didx, sem_di)

    def ld_wait(didx):
        pltpu.make_async_copy(dst.at[pl.ds(base_e, K)], didx, sem_di).wait()

    def deg_start(didx, core):
        if sh_deg is not None:
            @pl.when(c == core)
            def _():
                pltpu.async_copy(ones, sh_deg.at[didx], sem_o, add=True)

    def deg_wait(didx, core):
        if sh_deg is not None:
            @pl.when(c == core)
            def _():
                pltpu.make_async_copy(ones, sh_deg.at[didx], sem_o).wait()

    li_start(0, si0)
    ld_start(0, di0)
    li_wait(si0)
    pltpu.async_copy(h.at[si0], rows0, sem_g)
    li_start(1, si1)

    NJ = NCH // 2

    def pair(j, _):
        pltpu.make_async_copy(h.at[si0], rows0, sem_g).wait()
        ld_wait(di0)
        pltpu.async_copy(rows0, sh_S.at[di0], sem_s, add=True)
        deg_start(di0, 0)

        @pl.when(j > 0)
        def _():
            pltpu.make_async_copy(rows1, sh_S.at[di1], sem_s).wait()
            deg_wait(di1, 1)

        ld_start(2 * j + 1, di1)
        li_wait(si1)
        pltpu.async_copy(h.at[si1], rows1, sem_g)

        @pl.when(j < NJ - 1)
        def _():
            li_start(2 * j + 2, si0)

        pltpu.make_async_copy(h.at[si1], rows1, sem_g).wait()
        ld_wait(di1)
        pltpu.async_copy(rows1, sh_S.at[di1], sem_s, add=True)
        deg_start(di1, 1)

        pltpu.make_async_copy(rows0, sh_S.at[di0], sem_s).wait()
        deg_wait(di0, 0)

        @pl.when(j < NJ - 1)
        def _():
            ld_start(2 * j + 2, di0)
            li_wait(si0)
            pltpu.async_copy(h.at[si0], rows0, sem_g)
            li_start(2 * j + 3, si1)

        return 0

    lax.fori_loop(0, NJ, pair, 0)
    pltpu.make_async_copy(rows1, sh_S.at[di1], sem_s).wait()
    deg_wait(di1, 1)


def _gather_rows(table, idx_hbm, out_hbm, gidx, grows, sem, wid):
    gb = wid * GPT
    pltpu.sync_copy(idx_hbm.at[pl.ds(gb, GPT)], gidx)
    pltpu.async_copy(table.at[gidx], grows, sem).wait()
    pltpu.sync_copy(grows, out_hbm.at[pl.ds(gb, GPT)])


_EDGE_SCRATCH = [
    pltpu.VMEM((K,), jnp.int32),
    pltpu.VMEM((K,), jnp.int32),
    pltpu.VMEM((K, H), jnp.float32),
    pltpu.VMEM((K,), jnp.int32),
    pltpu.VMEM((K,), jnp.int32),
    pltpu.VMEM((K, H), jnp.float32),
    pltpu.SemaphoreType.DMA,
    pltpu.SemaphoreType.DMA,
    pltpu.SemaphoreType.DMA,
    pltpu.SemaphoreType.DMA,
]


@functools.partial(
    pl.kernel,
    out_type=(
        jax.ShapeDtypeStruct((N_USER, D), jnp.float32),
        jax.ShapeDtypeStruct((NC * N_USER,), jnp.float32),
    ),
    mesh=plsc.VectorSubcoreMesh(**_MESH),
    compiler_params=pltpu.CompilerParams(use_tc_tiling_on_sc=False),
    scratch_types=_EDGE_SCRATCH + [
        pltpu.VMEM((K,), jnp.float32),
        pltpu.VMEM((N_USER,), jnp.float32),
        pltpu.VMEM_SHARED((N_USER, H), jnp.float32),
        pltpu.VMEM_SHARED((N_USER,), jnp.float32),
        pltpu.SemaphoreType.DMA,
    ],
)
def _sc_layer1(hL, hR, src, dst, zf,
               S_out, deg_out,
               si0, di0, rows0, si1, di1, rows1,
               sem_g, sem_s, sem_si, sem_di,
               ones, deg_v, sh_S, sh_deg, sem_o):
    c = lax.axis_index("c")
    s = lax.axis_index("s")
    co = c * H

    _zero_stripe(zf, sh_S, s)

    @pl.when(s == 0)
    def _():
        def zbody(i, _):
            deg_v[pl.ds(i * 16, 16)] = jnp.zeros((16,), jnp.float32)
            return 0
        lax.fori_loop(0, N_USER // 16, zbody, 0)
        pltpu.sync_copy(deg_v, sh_deg)

    _fill_ones(ones)
    plsc.subcore_barrier()

    @pl.when(c == 0)
    def _():
        _edge_pipeline(hL, src, dst, s, c, sh_S,
                       sem_g, sem_s, sem_si, sem_di,
                       si0, di0, rows0, si1, di1, rows1,
                       ones=ones, sh_deg=sh_deg, sem_o=sem_o)

    @pl.when(c == 1)
    def _():
        _edge_pipeline(hR, src, dst, s, c, sh_S,
                       sem_g, sem_s, sem_si, sem_di,
                       si0, di0, rows0, si1, di1, rows1,
                       ones=ones, sh_deg=sh_deg, sem_o=sem_o)

    plsc.subcore_barrier()
    _write_stripes(sh_S, S_out, s, co)

    @pl.when(s == 0)
    def _():
        pltpu.sync_copy(sh_deg, deg_v)
        pltpu.sync_copy(deg_v, deg_out.at[pl.ds(c * N_USER, N_USER)])


@functools.partial(
    pl.kernel,
    out_type=(
        jax.ShapeDtypeStruct((N_USER, D), jnp.float32),
        jax.ShapeDtypeStruct((B, D), jnp.float32),
    ),
    mesh=plsc.VectorSubcoreMesh(**_MESH),
    compiler_params=pltpu.CompilerParams(use_tc_tiling_on_sc=False),
    scratch_types=_EDGE_SCRATCH + [
        pltpu.VMEM((GPT,), jnp.int32),
        pltpu.VMEM((GPT, D), jnp.float32),
        pltpu.VMEM_SHARED((N_USER, H), jnp.float32),
        pltpu.SemaphoreType.DMA,
    ],
)
def _sc_layer2(h_full, hL, hR, src, dst, users, zf,
               S_out, gu,
               si0, di0, rows0, si1, di1, rows1,
               sem_g, sem_s, sem_si, sem_di,
               gidx, grows, sh_S, sem_u):
    c = lax.axis_index("c")
    s = lax.axis_index("s")
    co = c * H
    wid = c * NS + s

    _zero_stripe(zf, sh_S, s)
    plsc.subcore_barrier()

    @pl.when(c == 0)
    def _():
        _edge_pipeline(hL, src, dst, s, c, sh_S,
                       sem_g, sem_s, sem_si, sem_di,
                       si0, di0, rows0, si1, di1, rows1)

    @pl.when(c == 1)
    def _():
        _edge_pipeline(hR, src, dst, s, c, sh_S,
                       sem_g, sem_s, sem_si, sem_di,
                       si0, di0, rows0, si1, di1, rows1)

    _gather_rows(h_full, users, gu, gidx, grows, sem_u, wid)

    plsc.subcore_barrier()
    _write_stripes(sh_S, S_out, s, co)


@functools.partial(
    pl.kernel,
    out_type=jax.ShapeDtypeStruct((B, D), jnp.float32),
    mesh=plsc.VectorSubcoreMesh(**_MESH),
    scratch_types=[
        pltpu.VMEM((GPT,), jnp.int32),
        pltpu.VMEM((GPT, D), jnp.float32),
        pltpu.SemaphoreType.DMA,
    ],
)
def _sc_gather(table, idx, out, gidx, grows, sem):
    c = lax.axis_index("c")
    s = lax.axis_index("s")
    wid = c * NS + s
    _gather_rows(table, idx, out, gidx, grows, sem, wid)


@functools.partial(
    pl.kernel,
    out_type=(
        jax.ShapeDtypeStruct((B, D), jnp.float32),
        jax.ShapeDtypeStruct((B, D), jnp.float32),
        jax.ShapeDtypeStruct((B, D), jnp.float32),
    ),
    mesh=plsc.VectorSubcoreMesh(**_MESH),
    scratch_types=[
        pltpu.VMEM((GPT,), jnp.int32),
        pltpu.VMEM((GPT, D), jnp.float32),
        pltpu.SemaphoreType.DMA,
    ],
)
def _sc_gather3(fuser, fitem, users, pos, neg, gu, gp, gn, gidx, grows, sem):
    c = lax.axis_index("c")
    s = lax.axis_index("s")
    wid = c * NS + s
    _gather_rows(fuser, users, gu, gidx, grows, sem, wid)
    _gather_rows(fitem, pos, gp, gidx, grows, sem, wid)
    _gather_rows(fitem, neg, gn, gidx, grows, sem, wid)


def _leaky_l2(z):
    z = jnp.where(z >= 0, z, 0.2 * z)
    n2 = jnp.sum(z * z, axis=1, keepdims=True)
    return z * lax.rsqrt(jnp.maximum(n2, 1e-24))


def _sage_z(h_ref, S_ref, deg_ref, Ws_ref, Wn_ref, b_ref):
    deg = jnp.sum(deg_ref[...], axis=1, keepdims=True)
    inv = 1.0 / jnp.maximum(deg, 1.0)
    z = (jnp.dot(h_ref[...], Ws_ref[...], preferred_element_type=jnp.float32)
         + jnp.dot(S_ref[...] * inv, Wn_ref[...],
                   preferred_element_type=jnp.float32)
         + b_ref[...])
    return _leaky_l2(z)


def _sage_body3(h_ref, S_ref, deg_ref, Ws_ref, Wn_ref, b_ref,
                o_ref, oL_ref, oR_ref):
    z = _sage_z(h_ref, S_ref, deg_ref, Ws_ref, Wn_ref, b_ref)
    o_ref[...] = z
    oL_ref[...] = z[:, :H]
    oR_ref[...] = z[:, H:]


def _sage_body1(h_ref, S_ref, deg_ref, Ws_ref, Wn_ref, b_ref, o_ref):
    o_ref[...] = _sage_z(h_ref, S_ref, deg_ref, Ws_ref, Wn_ref, b_ref)


_SAGE_RB = 2000

_SAGE_IN_SPECS = [
    pl.BlockSpec((_SAGE_RB, D), lambda i: (i, 0)),
    pl.BlockSpec((_SAGE_RB, D), lambda i: (i, 0)),
    pl.BlockSpec((_SAGE_RB, NC), lambda i: (i, 0)),
    pl.BlockSpec((D, D), lambda i: (0, 0)),
    pl.BlockSpec((D, D), lambda i: (0, 0)),
    pl.BlockSpec((1, D), lambda i: (0, 0)),
]


def _sage_tc(h, S, deg2, Ws, Wn, b2, with_halves):
    if with_halves:
        return pl.pallas_call(
            _sage_body3,
            grid=(N_USER // _SAGE_RB,),
            in_specs=_SAGE_IN_SPECS,
            out_specs=[
                pl.BlockSpec((_SAGE_RB, D), lambda i: (i, 0)),
                pl.BlockSpec((_SAGE_RB, H), lambda i: (i, 0)),
                pl.BlockSpec((_SAGE_RB, H), lambda i: (i, 0)),
            ],
            out_shape=[
                jax.ShapeDtypeStruct((N_USER, D), jnp.float32),
                jax.ShapeDtypeStruct((N_USER, H), jnp.float32),
                jax.ShapeDtypeStruct((N_USER, H), jnp.float32),
            ],
        )(h, S, deg2, Ws, Wn, b2)
    return pl.pallas_call(
        _sage_body1,
        grid=(N_USER // _SAGE_RB,),
        in_specs=_SAGE_IN_SPECS,
        out_specs=pl.BlockSpec((_SAGE_RB, D), lambda i: (i, 0)),
        out_shape=jax.ShapeDtypeStruct((N_USER, D), jnp.float32),
    )(h, S, deg2, Ws, Wn, b2)


def _item_body(x_ref, W0_ref, b0_ref, W1_ref, b1_ref, o_ref):
    x0 = x_ref[...]
    x1 = _leaky_l2(jnp.dot(x0, W0_ref[...],
                           preferred_element_type=jnp.float32) + b0_ref[...])
    x2 = _leaky_l2(jnp.dot(x1, W1_ref[...],
                           preferred_element_type=jnp.float32) + b1_ref[...])
    o_ref[:, 0:D] = x0
    o_ref[:, D:2 * D] = x1
    o_ref[:, 2 * D:3 * D] = x2


_ITEM_RB = 1024


def _item_tc(x0, W0, b0, W1, b1):
    return pl.pallas_call(
        _item_body,
        grid=(B // _ITEM_RB,),
        in_specs=[
            pl.BlockSpec((_ITEM_RB, D), lambda i: (i, 0)),
            pl.BlockSpec((D, D), lambda i: (0, 0)),
            pl.BlockSpec((1, D), lambda i: (0, 0)),
            pl.BlockSpec((D, D), lambda i: (0, 0)),
            pl.BlockSpec((1, D), lambda i: (0, 0)),
        ],
        out_specs=pl.BlockSpec((_ITEM_RB, 3 * D), lambda i: (i, 0)),
        out_shape=jax.ShapeDtypeStruct((B, 3 * D), jnp.float32),
    )(x0, W0, b0, W1, b1)


def kernel(users, pos_items, neg_items, edge_index, feat_user, feat_item,
           W_self_0, W_neigh_0, b_0, W_self_1, W_neigh_1, b_1,
           Wi_0, bi_0, Wi_1, bi_1):
    src = edge_index[0].astype(jnp.int32)
    dst = edge_index[1].astype(jnp.int32)
    users = users.astype(jnp.int32)
    pos_items = pos_items.astype(jnp.int32)
    neg_items = neg_items.astype(jnp.int32)

    zf = jnp.zeros((STRIPE_LAST, H), jnp.float32)

    S0, deg = _sc_layer1(feat_user[:, :H], feat_user[:, H:], src, dst, zf)
    gu0, gp0, gn0 = _sc_gather3(feat_user, feat_item, users, pos_items,
                                neg_items)
    deg2 = deg.reshape(NC, N_USER).T

    h1, h1L, h1R = _sage_tc(feat_user, S0, deg2,
                            W_self_0, W_neigh_0, b_0.reshape(1, D), True)
    S1, gu1 = _sc_layer2(h1, h1L, h1R, src, dst, users, zf)
    h2 = _sage_tc(h1, S1, deg2, W_self_1, W_neigh_1, b_1.reshape(1, D), False)
    gu2 = _sc_gather(h2, users)

    pos_i_g = _item_tc(gp0, Wi_0, bi_0.reshape(1, D), Wi_1, bi_1.reshape(1, D))
    neg_i_g = _item_tc(gn0, Wi_0, bi_0.reshape(1, D), Wi_1, bi_1.reshape(1, D))

    u_g = jnp.concatenate([gu0, gu1, gu2], axis=1)
    return (u_g, pos_i_g, neg_i_g)
